# baseline (speedup 1.0000x reference)
import functools

import jax
import jax.numpy as jnp
from jax import lax
from jax.experimental import pallas as pl
from jax.experimental.pallas import tpu as pltpu
from jax.experimental.pallas import tpu_sc as plsc

N = 160000
B = 8
M = 500
D = 8
NC = 2
NSUB = 16
NW = NC * NSUB
CHUNK = N // NSUB
PPW = (B * M) // NW
PPAD = 128
NV = PPW * D // 16 + 1
WV = PPW * D


def _body(output_hbm, bidx_hbm, pk_hbm, tgt_hbm, out_hbm,
          bi_v, cnt_v, all_v, starts_v, pk_v, tgt_v,
          row_v, pred_v, part_v, acc_v, out_stage,
          sall_sh, sacc_sh, sem, sem_pk, sem_tgt):
    c = lax.axis_index("c")
    s = lax.axis_index("s")
    wid = s * NC + c
    iota = lax.iota(jnp.int32, 16)
    b = wid >> 2
    q = wid & 3

    tgt_cp = pltpu.async_copy(tgt_hbm.at[b, pl.ds(q * WV, WV)],
                              tgt_v.at[pl.ds(0, WV)], sem_tgt)
    pk_cp = pltpu.async_copy(pk_hbm.at[wid], pk_v, sem_pk)
    tgt_v[pl.ds(WV, 16)] = jnp.zeros((16,), jnp.float32)

    pltpu.sync_copy(bidx_hbm.at[pl.ds(s * CHUNK, CHUNK)], bi_v)
    pos = jnp.zeros((16,), jnp.int32)
    step = 8192
    while step:
        npos = pos + step
        probe = jnp.minimum(npos, CHUNK) - 1
        v = plsc.load_gather(bi_v, [probe])
        take = (npos <= CHUNK) & (v < iota)
        pos = jnp.where(take, npos, pos)
        step >>= 1
    cnt_v[...] = pos
    pltpu.sync_copy(cnt_v, sall_sh.at[s])
    plsc.subcore_barrier()
    pltpu.sync_copy(sall_sh, all_v)

    def _sum_starts(r, a):
        return a + all_v[r]

    starts_v[...] = lax.fori_loop(1, NSUB, _sum_starts, all_v[0])

    pk_cp.wait()
    start_b = plsc.load_gather(starts_v, [jnp.full((16,), 0, jnp.int32) + b])

    def _rows(k, a):
        pk = pk_v[pl.ds(k * 16, 16)]
        iv = (pk & 0xFFFFF) + start_b
        row_v[pl.ds(k * 16, 16)] = jnp.minimum(jnp.maximum(iv, 0), N - 1)
        return a + (pk >> 20).astype(jnp.float32)

    accn = lax.fori_loop(0, PPAD // 16, _rows,
                         jnp.zeros((16,), jnp.float32))

    ge8 = iota >> 3
    col = iota & 7

    def _gather(k, a):
        row2 = plsc.load_gather(row_v, [ge8 + 2 * k])
        fidx = ((row2 >> 7) << 10) + (col << 7) + (row2 & 127)
        pltpu.async_copy(output_hbm.at[fidx],
                         pred_v.at[pl.ds(k * 16, 16)], sem)
        return a

    lax.fori_loop(0, NV, _gather, 0)
    tgt_cp.wait()
    pltpu.make_async_copy(output_hbm.at[pl.ds(0, NV * 16)],
                          pred_v.at[pl.ds(0, NV * 16)], sem).wait()

    def _loss(k, a):
        p = pred_v[pl.ds(k * 16, 16)]
        g = plsc.load_gather(pk_v, [ge8 + 2 * k])
        t = tgt_v[pl.ds(k * 16, 16)]
        m = (g >> 20).astype(jnp.float32) * jnp.where(t != t, 0.0, 1.0)
        return a + jnp.abs(p * m - t * m)

    acc = lax.fori_loop(0, NV, _loss, jnp.zeros((16,), jnp.float32))

    part_v[0] = acc
    part_v[1] = accn
    pltpu.sync_copy(part_v, sacc_sh.at[s])
    plsc.subcore_barrier()

    @pl.when(s == 0)
    def _():
        pltpu.sync_copy(sacc_sh, acc_v)

        def _comb(r, la_na):
            la, na = la_na
            return la + acc_v[r, 0], na + acc_v[r, 1]

        lacc, nacc = lax.fori_loop(1, NSUB, _comb, (acc_v[0, 0], acc_v[0, 1]))
        num = jnp.sum(nacc)
        out_stage[...] = lacc
        hi = plsc.load_gather(out_stage, [(iota & 7) + 8])
        out_stage[...] = jnp.where(iota < 8, lacc + hi, num)
        pltpu.sync_copy(out_stage, out_hbm.at[c])


_call = functools.partial(
    pl.kernel,
    out_type=jax.ShapeDtypeStruct((NC, 16), jnp.float32),
    mesh=plsc.VectorSubcoreMesh(core_axis_name="c", subcore_axis_name="s",
                                num_cores=NC),
    compiler_params=pltpu.CompilerParams(needs_layout_passes=False,
                                         use_tc_tiling_on_sc=False,
                                         skip_device_barrier=True),
    scratch_types=[
        pltpu.VMEM((CHUNK,), jnp.int32),
        pltpu.VMEM((16,), jnp.int32),
        pltpu.VMEM((NSUB, 16), jnp.int32),
        pltpu.VMEM((16,), jnp.int32),
        pltpu.VMEM((PPAD,), jnp.int32),
        pltpu.VMEM((1024,), jnp.float32),
        pltpu.VMEM((PPAD,), jnp.int32),
        pltpu.VMEM((1024,), jnp.float32),
        pltpu.VMEM((2, 16), jnp.float32),
        pltpu.VMEM((NSUB, 2, 16), jnp.float32),
        pltpu.VMEM((16,), jnp.float32),
        pltpu.VMEM_SHARED((NSUB, 16), jnp.int32),
        pltpu.VMEM_SHARED((NSUB, 2, 16), jnp.float32),
        pltpu.SemaphoreType.DMA,
        pltpu.SemaphoreType.DMA,
        pltpu.SemaphoreType.DMA,
    ],
)(_body)


def _tc_combine(in_ref, out_ref):
    x = in_ref[0] + in_ref[1]
    out_ref[...] = x[0:8] / jnp.maximum(x[8:16], 1.0)


_tc_call = pl.pallas_call(
    _tc_combine,
    out_shape=jax.ShapeDtypeStruct((D,), jnp.float32),
)


def kernel(output, mask, ind, target, batch_index):
    bidx = batch_index.astype(jnp.int32)
    out_lin = output.T.reshape(D, N // 128, 128).transpose(1, 0, 2).reshape(-1)
    pad = ((0, 0), (0, PPAD - PPW))
    packed = ind.astype(jnp.int32) | (mask.astype(jnp.int32) << 20)
    pk32 = jnp.pad(packed.reshape(NW, PPW), pad)
    tgt8 = target.reshape(B, M * D)
    parts = _call(out_lin, bidx, pk32, tgt8)
    return _tc_call(parts)


# speedup vs baseline: 1.0477x; 1.0477x over previous
import functools

import jax
import jax.numpy as jnp
from jax import lax
from jax.experimental import pallas as pl
from jax.experimental.pallas import tpu as pltpu
from jax.experimental.pallas import tpu_sc as plsc

N = 160000
B = 8
M = 500
D = 8
NC = 2
NSUB = 16
NW = NC * NSUB
CHUNK = N // NSUB
PPW = (B * M) // NW
PPAD = 128
NV = PPW * D // 16 + 1
WV = PPW * D


def _body(output_hbm, bidx_hbm, pk_hbm, tgt_hbm, out_hbm,
          bi_v, cnt_v, all_v, starts_v, pk_v, tgt_v,
          row_v, pred_v, part_v, acc_v, out_stage,
          sall_sh, sacc_sh, sem, sem_pk, sem_tgt):
    c = lax.axis_index("c")
    s = lax.axis_index("s")
    wid = s * NC + c
    iota = lax.iota(jnp.int32, 16)
    b = wid >> 2
    q = wid & 3

    tgt_cp = pltpu.async_copy(tgt_hbm.at[b, pl.ds(q * WV, WV)],
                              tgt_v.at[pl.ds(0, WV)], sem_tgt)
    pk_cp = pltpu.async_copy(pk_hbm.at[wid], pk_v, sem_pk)
    tgt_v[pl.ds(WV, 16)] = jnp.zeros((16,), jnp.float32)

    pltpu.sync_copy(bidx_hbm.at[pl.ds(s * CHUNK, CHUNK)], bi_v)
    pos = jnp.zeros((16,), jnp.int32)
    step = 8192
    while step:
        npos = pos + step
        probe = jnp.minimum(npos, CHUNK) - 1
        v = plsc.load_gather(bi_v, [probe])
        take = (npos <= CHUNK) & (v < iota)
        pos = jnp.where(take, npos, pos)
        step >>= 1
    cnt_v[...] = pos
    pltpu.sync_copy(cnt_v, sall_sh.at[s])
    plsc.subcore_barrier()
    pltpu.sync_copy(sall_sh, all_v)

    def _sum_starts(r, a):
        return a + all_v[r]

    starts_v[...] = lax.fori_loop(1, NSUB, _sum_starts, all_v[0])

    pk_cp.wait()
    start_b = plsc.load_gather(starts_v, [jnp.full((16,), 0, jnp.int32) + b])

    def _rows(k, a):
        pk = pk_v[pl.ds(k * 16, 16)]
        iv = (pk & 0xFFFFF) + start_b
        row_v[pl.ds(k * 16, 16)] = jnp.minimum(jnp.maximum(iv, 0), N - 1)
        return a + (pk >> 20).astype(jnp.float32)

    accn = lax.fori_loop(0, PPAD // 16, _rows,
                         jnp.zeros((16,), jnp.float32))

    ge8 = iota >> 3
    col = iota & 7

    def _gather(k, a):
        row2 = plsc.load_gather(row_v, [ge8 + 2 * k])
        fidx = ((row2 >> 7) << 10) + (col << 7) + (row2 & 127)
        pltpu.async_copy(output_hbm.at[fidx],
                         pred_v.at[pl.ds(k * 16, 16)], sem)
        return a

    lax.fori_loop(0, NV, _gather, 0)
    tgt_cp.wait()
    pltpu.make_async_copy(output_hbm.at[pl.ds(0, NV * 16)],
                          pred_v.at[pl.ds(0, NV * 16)], sem).wait()

    def _loss(k, a):
        p = pred_v[pl.ds(k * 16, 16)]
        g = plsc.load_gather(pk_v, [ge8 + 2 * k])
        t = tgt_v[pl.ds(k * 16, 16)]
        m = (g >> 20).astype(jnp.float32) * jnp.where(t != t, 0.0, 1.0)
        return a + jnp.abs(p * m - t * m)

    acc = lax.fori_loop(0, NV, _loss, jnp.zeros((16,), jnp.float32))

    part_v[0] = acc
    part_v[1] = accn
    pltpu.sync_copy(part_v, sacc_sh.at[s])
    plsc.subcore_barrier()

    @pl.when(s == 0)
    def _():
        pltpu.sync_copy(sacc_sh, acc_v)

        def _comb(r, la_na):
            la, na = la_na
            return la + acc_v[r, 0], na + acc_v[r, 1]

        lacc, nacc = lax.fori_loop(1, NSUB, _comb, (acc_v[0, 0], acc_v[0, 1]))
        num = jnp.sum(nacc)
        out_stage[...] = lacc
        hi = plsc.load_gather(out_stage, [(iota & 7) + 8])
        out_stage[...] = jnp.where(iota < 8, lacc + hi, num)
        pltpu.sync_copy(out_stage, out_hbm.at[c, pl.ds(0, 16)])


_call = functools.partial(
    pl.kernel,
    out_type=jax.ShapeDtypeStruct((8, 128), jnp.float32),
    mesh=plsc.VectorSubcoreMesh(core_axis_name="c", subcore_axis_name="s",
                                num_cores=NC),
    compiler_params=pltpu.CompilerParams(needs_layout_passes=False,
                                         use_tc_tiling_on_sc=False,
                                         skip_device_barrier=True),
    scratch_types=[
        pltpu.VMEM((CHUNK,), jnp.int32),
        pltpu.VMEM((16,), jnp.int32),
        pltpu.VMEM((NSUB, 16), jnp.int32),
        pltpu.VMEM((16,), jnp.int32),
        pltpu.VMEM((PPAD,), jnp.int32),
        pltpu.VMEM((1024,), jnp.float32),
        pltpu.VMEM((PPAD,), jnp.int32),
        pltpu.VMEM((1024,), jnp.float32),
        pltpu.VMEM((2, 16), jnp.float32),
        pltpu.VMEM((NSUB, 2, 16), jnp.float32),
        pltpu.VMEM((16,), jnp.float32),
        pltpu.VMEM_SHARED((NSUB, 16), jnp.int32),
        pltpu.VMEM_SHARED((NSUB, 2, 16), jnp.float32),
        pltpu.SemaphoreType.DMA,
        pltpu.SemaphoreType.DMA,
        pltpu.SemaphoreType.DMA,
    ],
)(_body)


def _tc_combine(in_ref, out_ref):
    x = in_ref[0, pl.ds(0, 16)] + in_ref[1, pl.ds(0, 16)]
    out_ref[...] = x[0:8] / jnp.maximum(x[8:16], 1.0)


_tc_call = pl.pallas_call(
    _tc_combine,
    out_shape=jax.ShapeDtypeStruct((D,), jnp.float32),
)


def kernel(output, mask, ind, target, batch_index):
    bidx = batch_index.astype(jnp.int32)
    out_lin = output.T.reshape(D, N // 128, 128).transpose(1, 0, 2).reshape(-1)
    pad = ((0, 0), (0, PPAD - PPW))
    packed = ind.astype(jnp.int32) | (mask.astype(jnp.int32) << 20)
    pk32 = jnp.pad(packed.reshape(NW, PPW), pad)
    tgt8 = target.reshape(B, M * D)
    parts = _call(out_lin, bidx, pk32, tgt8)
    return _tc_call(parts)


# speedup vs baseline: 1.1538x; 1.1013x over previous
import functools

import jax
import jax.numpy as jnp
from jax import lax
from jax.experimental import pallas as pl
from jax.experimental.pallas import tpu as pltpu
from jax.experimental.pallas import tpu_sc as plsc

N = 160000
B = 8
M = 500
D = 8
NSUB = 16
CHUNK = N // NSUB
MP = 512
NCH = M // 16
TAIL = M - 16
DG = 4


def _body(output_hbm, bidx_hbm, pk_hbm, tgt_hbm, out_hbm,
          bi_v, cnt_v, all_v, starts_v, pk_v, tgt_v,
          row_v, pred_v, part_v, acc_v, out_stage,
          sall_sh, sacc_sh, sem, sem_pk, sem_tgt):
    w = lax.axis_index("s")
    iota = lax.iota(jnp.int32, 16)
    b = w >> 1
    h = w & 1

    tgt_cp = pltpu.async_copy(tgt_hbm.at[b, pl.ds(DG * M * h, DG * M)],
                              tgt_v, sem_tgt)
    pk_cp = pltpu.async_copy(pk_hbm.at[b], pk_v, sem_pk)

    pltpu.sync_copy(bidx_hbm.at[pl.ds(w * CHUNK, CHUNK)], bi_v)
    pos = jnp.zeros((16,), jnp.int32)
    step = 8192
    while step:
        npos = pos + step
        probe = jnp.minimum(npos, CHUNK) - 1
        v = plsc.load_gather(bi_v, [probe])
        take = (npos <= CHUNK) & (v < iota)
        pos = jnp.where(take, npos, pos)
        step >>= 1
    cnt_v[...] = pos
    pltpu.sync_copy(cnt_v, sall_sh.at[w])
    plsc.subcore_barrier()
    pltpu.sync_copy(sall_sh, all_v)

    def _sum_starts(r, a):
        return a + all_v[r]

    starts_v[...] = lax.fori_loop(1, NSUB, _sum_starts, all_v[0])

    pk_cp.wait()
    start_b = plsc.load_gather(starts_v, [jnp.full((16,), 0, jnp.int32) + b])

    def _rows(k, a):
        pk = pk_v[pl.ds(k * 16, 16)]
        iv = (pk & 0xFFFFF) + start_b
        row_v[pl.ds(k * 16, 16)] = jnp.minimum(jnp.maximum(iv, 0), N - 1)
        return a + (pk >> 20).astype(jnp.float32)

    accn = lax.fori_loop(0, MP // 16, _rows, jnp.zeros((16,), jnp.float32))
    accn = jnp.where(h == 0, accn, 0.0)

    for dd in range(DG):
        d_off = (DG * h + dd) << 7

        def _gather(kc, a, _dd=dd, _d_off=d_off):
            r16 = row_v[pl.ds(kc * 16, 16)]
            fidx = ((r16 >> 7) << 10) + _d_off + (r16 & 127)
            pltpu.async_copy(output_hbm.at[fidx],
                             pred_v.at[pl.ds(_dd * MP + kc * 16, 16)], sem)
            return a

        lax.fori_loop(0, NCH, _gather, 0)
        r16 = row_v[pl.ds(TAIL, 16)]
        fidx = ((r16 >> 7) << 10) + d_off + (r16 & 127)
        pltpu.async_copy(output_hbm.at[fidx],
                         pred_v.at[pl.ds(dd * MP + 496, 16)], sem)

    tgt_cp.wait()
    for _ in range(DG):
        pltpu.make_async_copy(output_hbm.at[pl.ds(0, (NCH + 1) * 16)],
                              pred_v.at[pl.ds(0, (NCH + 1) * 16)],
                              sem).wait()

    wtail = jnp.where(iota < 12, 0.0, 1.0)
    part = jnp.zeros((16,), jnp.float32)
    for dd in range(DG):
        def _loss(kc, a, _dd=dd):
            p = pred_v[pl.ds(_dd * MP + kc * 16, 16)]
            t = tgt_v[pl.ds(_dd * M + kc * 16, 16)]
            mk = (pk_v[pl.ds(kc * 16, 16)] >> 20).astype(jnp.float32)
            m = mk * jnp.where(t != t, 0.0, 1.0)
            return a + jnp.abs(p * m - t * m)

        acc_d = lax.fori_loop(0, NCH, _loss, jnp.zeros((16,), jnp.float32))
        p = pred_v[pl.ds(dd * MP + 496, 16)]
        t = tgt_v[pl.ds(dd * M + TAIL, 16)]
        mk = (pk_v[pl.ds(TAIL, 16)] >> 20).astype(jnp.float32)
        m = wtail * mk * jnp.where(t != t, 0.0, 1.0)
        acc_d = acc_d + jnp.abs(p * m - t * m)
        part = part + jnp.where(iota == (DG * h + dd), jnp.sum(acc_d), 0.0)

    part_v[0] = part
    part_v[1] = accn
    pltpu.sync_copy(part_v, sacc_sh.at[w])
    plsc.subcore_barrier()

    @pl.when(w == 0)
    def _():
        pltpu.sync_copy(sacc_sh, acc_v)

        def _comb(r, la_na):
            la, na = la_na
            return la + acc_v[r, 0], na + acc_v[r, 1]

        lacc, nacc = lax.fori_loop(1, NSUB, _comb, (acc_v[0, 0], acc_v[0, 1]))
        num = jnp.maximum(jnp.sum(nacc), 1.0)
        out_stage[...] = lacc / num
        pltpu.sync_copy(out_stage.at[pl.ds(0, D)], out_hbm)


_call = functools.partial(
    pl.kernel,
    out_type=jax.ShapeDtypeStruct((D,), jnp.float32),
    mesh=plsc.VectorSubcoreMesh(core_axis_name="c", subcore_axis_name="s",
                                num_cores=1),
    compiler_params=pltpu.CompilerParams(needs_layout_passes=False,
                                         use_tc_tiling_on_sc=False,
                                         skip_device_barrier=True),
    scratch_types=[
        pltpu.VMEM((CHUNK,), jnp.int32),
        pltpu.VMEM((16,), jnp.int32),
        pltpu.VMEM((NSUB, 16), jnp.int32),
        pltpu.VMEM((16,), jnp.int32),
        pltpu.VMEM((MP,), jnp.int32),
        pltpu.VMEM((DG * M,), jnp.float32),
        pltpu.VMEM((MP,), jnp.int32),
        pltpu.VMEM((DG * MP,), jnp.float32),
        pltpu.VMEM((2, 16), jnp.float32),
        pltpu.VMEM((NSUB, 2, 16), jnp.float32),
        pltpu.VMEM((16,), jnp.float32),
        pltpu.VMEM_SHARED((NSUB, 16), jnp.int32),
        pltpu.VMEM_SHARED((NSUB, 2, 16), jnp.float32),
        pltpu.SemaphoreType.DMA,
        pltpu.SemaphoreType.DMA,
        pltpu.SemaphoreType.DMA,
    ],
)(_body)


def kernel(output, mask, ind, target, batch_index):
    bidx = batch_index.astype(jnp.int32)
    out_lin = output.T.reshape(D, N // 128, 128).transpose(1, 0, 2).reshape(-1)
    packed = ind.astype(jnp.int32) | (mask.astype(jnp.int32) << 20)
    pk8 = jnp.pad(packed, ((0, 0), (0, MP - M)))
    tgt_t = target.transpose(0, 2, 1).reshape(B, D * M)
    return _call(out_lin, bidx, pk8, tgt_t)
